# Initial kernel scaffold; baseline (speedup 1.0000x reference)
import jax
import jax.numpy as jnp
from jax.experimental import pallas as pl


def kernel(dialogue, dialogue_lengths, visual_features, cumulative_dialogue, cumulative_lengths, num_questions, object_categories, object_bboxes, num_objects, category_emb, word_emb, W_vis, b_vis, W_ih, W_hh, b_ih, b_hh, W_out, b_out):
    raise NotImplementedError("write your pallas kernel here")


# speedup vs baseline: 4.1167x; 4.1167x over previous
import functools

import jax
import jax.numpy as jnp
from jax import lax
from jax.experimental import pallas as pl
from jax.experimental.pallas import tpu as pltpu
from jax.experimental.pallas import tpu_sc as plsc

B, T, O = 8, 512, 50
V, Dw, Dh, Dc, Dv, NC_CAT = 4900, 512, 512, 256, 2048, 91

SC_CORES, SC_SUBCORES = 2, 16
NW = SC_CORES * SC_SUBCORES

TB = 64
NT = T // TB


def _sc_gather(table, idx, n_rows, d):
    rpw = n_rows // NW
    mesh = plsc.VectorSubcoreMesh(
        core_axis_name="c", subcore_axis_name="s",
        num_cores=SC_CORES, num_subcores=SC_SUBCORES)

    @functools.partial(
        pl.kernel,
        mesh=mesh,
        out_type=jax.ShapeDtypeStruct((n_rows, d), jnp.float32),
        scratch_types=[
            pltpu.VMEM((rpw,), jnp.int32),
            pltpu.VMEM((rpw, d), jnp.float32),
            pltpu.SemaphoreType.DMA,
        ],
    )
    def k(table_hbm, idx_hbm, out_hbm, idx_v, rows_v, sem):
        wid = lax.axis_index("s") * SC_CORES + lax.axis_index("c")
        base = wid * rpw
        pltpu.sync_copy(idx_hbm.at[pl.ds(base, rpw)], idx_v)
        pltpu.async_copy(table_hbm.at[idx_v], rows_v, sem).wait()
        pltpu.sync_copy(rows_v, out_hbm.at[pl.ds(base, rpw)])

    return k(table, idx)


def _const_body(cats_ref, nobj_ref, cemb_ref, vf_ref, wvis_ref, bvis_ref,
                wih_vc_ref, bias_ref, out_ref):
    cats = cats_ref[...]
    iota_c = lax.broadcasted_iota(jnp.int32, (B, NC_CAT), 1)
    m = jnp.zeros((B, NC_CAT), jnp.float32)
    for o in range(O):
        m = m + (cats[:, o:o + 1] == iota_c).astype(jnp.float32)
    ssum = jnp.dot(m, cemb_ref[...], preferred_element_type=jnp.float32)
    add = ssum / nobj_ref[...]
    vis = jnp.tanh(
        jnp.dot(vf_ref[...], wvis_ref[...], preferred_element_type=jnp.float32)
        + bvis_ref[...])
    va = jnp.concatenate([vis, add], axis=1)
    out_ref[...] = (
        jnp.dot(va, wih_vc_ref[...], preferred_element_type=jnp.float32)
        + bias_ref[...])


def _main_body(w_ref, wih_w_ref, whh_ref, const_ref, wout_ref, bout_ref,
               out_ref, xw_s, hs_s, h_s, c_s):
    i = pl.program_id(0)

    @pl.when(i == 0)
    def _():
        h_s[...] = jnp.zeros_like(h_s)
        c_s[...] = jnp.zeros_like(c_s)

    xw_s[...] = jnp.dot(w_ref[...], wih_w_ref[...],
                        preferred_element_type=jnp.float32)

    whh = whh_ref[...]
    const = const_ref[...]

    def step(t, carry):
        h, c = carry
        g = (xw_s[pl.ds(t * B, B), :]
             + jnp.dot(h, whh, preferred_element_type=jnp.float32)
             + const)
        gi = jax.nn.sigmoid(g[:, :Dh])
        gf = jax.nn.sigmoid(g[:, Dh:2 * Dh])
        gg = jnp.tanh(g[:, 2 * Dh:3 * Dh])
        go = jax.nn.sigmoid(g[:, 3 * Dh:])
        c2 = gf * c + gi * gg
        h2 = go * jnp.tanh(c2)
        hs_s[pl.ds(t * B, B), :] = h2
        return h2, c2

    hN, cN = lax.fori_loop(0, TB, step, (h_s[...], c_s[...]))
    h_s[...] = hN
    c_s[...] = cN

    hs = hs_s[...].reshape(TB, B, Dh)
    hsb = jnp.transpose(hs, (1, 0, 2)).reshape(B * TB, Dh)
    logits = (jnp.dot(hsb, wout_ref[...], preferred_element_type=jnp.float32)
              + bout_ref[...])
    out_ref[...] = logits.reshape(B, TB, V)


def kernel(dialogue, dialogue_lengths, visual_features, cumulative_dialogue,
           cumulative_lengths, num_questions, object_categories, object_bboxes,
           num_objects, category_emb, word_emb, W_vis, b_vis, W_ih, W_hh,
           b_ih, b_hh, W_out, b_out):
    idx = dialogue.astype(jnp.int32).T.reshape(-1)
    w_tm = _sc_gather(word_emb, idx, T * B, Dw)

    const = pl.pallas_call(
        _const_body,
        out_shape=jax.ShapeDtypeStruct((B, 4 * Dh), jnp.float32),
    )(
        object_categories.astype(jnp.int32),
        num_objects.astype(jnp.float32).reshape(B, 1),
        category_emb,
        visual_features,
        W_vis,
        b_vis.reshape(1, Dh),
        W_ih[Dw:],
        (b_ih + b_hh).reshape(1, 4 * Dh),
    )

    grid = (NT,)
    out = pl.pallas_call(
        _main_body,
        grid=grid,
        in_specs=[
            pl.BlockSpec((TB * B, Dw), lambda i: (i, 0)),
            pl.BlockSpec((Dw, 4 * Dh), lambda i: (0, 0)),
            pl.BlockSpec((Dh, 4 * Dh), lambda i: (0, 0)),
            pl.BlockSpec((B, 4 * Dh), lambda i: (0, 0)),
            pl.BlockSpec((Dh, V), lambda i: (0, 0)),
            pl.BlockSpec((1, V), lambda i: (0, 0)),
        ],
        out_specs=pl.BlockSpec((B, TB, V), lambda i: (0, i, 0)),
        out_shape=jax.ShapeDtypeStruct((B, T, V), jnp.float32),
        scratch_shapes=[
            pltpu.VMEM((TB * B, 4 * Dh), jnp.float32),
            pltpu.VMEM((TB * B, Dh), jnp.float32),
            pltpu.VMEM((B, Dh), jnp.float32),
            pltpu.VMEM((B, Dh), jnp.float32),
        ],
    )(w_tm, W_ih[:Dw], W_hh, const, W_out, b_out.reshape(1, V))
    return out


# speedup vs baseline: 4.1214x; 1.0011x over previous
import functools

import jax
import jax.numpy as jnp
from jax import lax
from jax.experimental import pallas as pl
from jax.experimental.pallas import tpu as pltpu
from jax.experimental.pallas import tpu_sc as plsc

B, T, O = 8, 512, 50
V, Dw, Dh, Dc, Dv, NC_CAT = 4900, 512, 512, 256, 2048, 91

SC_CORES, SC_SUBCORES = 2, 16
NW = SC_CORES * SC_SUBCORES

TB = 64
NT = T // TB


def _sc_gather(table, idx, n_rows, d):
    rpw = n_rows // NW
    mesh = plsc.VectorSubcoreMesh(
        core_axis_name="c", subcore_axis_name="s",
        num_cores=SC_CORES, num_subcores=SC_SUBCORES)

    @functools.partial(
        pl.kernel,
        mesh=mesh,
        out_type=jax.ShapeDtypeStruct((n_rows, d), jnp.float32),
        scratch_types=[
            pltpu.VMEM((rpw,), jnp.int32),
            pltpu.VMEM((rpw, d), jnp.float32),
            pltpu.SemaphoreType.DMA,
        ],
    )
    def k(table_hbm, idx_hbm, out_hbm, idx_v, rows_v, sem):
        wid = lax.axis_index("s") * SC_CORES + lax.axis_index("c")
        base = wid * rpw
        pltpu.sync_copy(idx_hbm.at[pl.ds(base, rpw)], idx_v)
        pltpu.async_copy(table_hbm.at[idx_v], rows_v, sem).wait()
        pltpu.sync_copy(rows_v, out_hbm.at[pl.ds(base, rpw)])

    return k(table, idx)


def _const_body(cats_ref, nobj_ref, cemb_ref, vf_ref, wvis_ref, bvis_ref,
                wih_vc_ref, bias_ref, out_ref):
    cats = cats_ref[...]
    iota_c = lax.broadcasted_iota(jnp.int32, (B, NC_CAT), 1)
    m = jnp.zeros((B, NC_CAT), jnp.float32)
    for o in range(O):
        m = m + (cats[:, o:o + 1] == iota_c).astype(jnp.float32)
    ssum = jnp.dot(m, cemb_ref[...], preferred_element_type=jnp.float32)
    add = ssum / nobj_ref[...]
    vis = jnp.tanh(
        jnp.dot(vf_ref[...], wvis_ref[...], preferred_element_type=jnp.float32)
        + bvis_ref[...])
    va = jnp.concatenate([vis, add], axis=1)
    out_ref[...] = (
        jnp.dot(va, wih_vc_ref[...], preferred_element_type=jnp.float32)
        + bias_ref[...])


def _main_body(w_ref, wih_w_ref, whh_ref, const_ref, wout_ref, bout_ref,
               out_ref, xw_s, hs_s, h_s, c_s):
    i = pl.program_id(0)

    @pl.when(i == 0)
    def _():
        h_s[...] = jnp.zeros_like(h_s)
        c_s[...] = jnp.zeros_like(c_s)

    xw_s[...] = (jnp.dot(w_ref[...].astype(jnp.bfloat16), wih_w_ref[...],
                         preferred_element_type=jnp.float32)
                 + jnp.tile(const_ref[...], (TB, 1)))

    whh = whh_ref[...]

    def step(t, carry):
        h, c = carry
        g = (xw_s[pl.ds(t * B, B), :]
             + jnp.dot(h.astype(jnp.bfloat16), whh,
                       preferred_element_type=jnp.float32))
        gi = jax.nn.sigmoid(g[:, :Dh])
        gf = jax.nn.sigmoid(g[:, Dh:2 * Dh])
        gg = jnp.tanh(g[:, 2 * Dh:3 * Dh])
        go = jax.nn.sigmoid(g[:, 3 * Dh:])
        c2 = gf * c + gi * gg
        h2 = go * jnp.tanh(c2)
        hs_s[pl.ds(t * B, B), :] = h2.astype(jnp.bfloat16)
        return h2, c2

    hN, cN = lax.fori_loop(0, TB, step, (h_s[...], c_s[...]))
    h_s[...] = hN
    c_s[...] = cN

    hs = hs_s[...].reshape(TB, B, Dh)
    hsb = jnp.transpose(hs, (1, 0, 2)).reshape(B * TB, Dh)
    logits = (jnp.dot(hsb, wout_ref[...], preferred_element_type=jnp.float32)
              + bout_ref[...])
    out_ref[...] = logits.reshape(B, TB, V)


def kernel(dialogue, dialogue_lengths, visual_features, cumulative_dialogue,
           cumulative_lengths, num_questions, object_categories, object_bboxes,
           num_objects, category_emb, word_emb, W_vis, b_vis, W_ih, W_hh,
           b_ih, b_hh, W_out, b_out):
    idx = dialogue.astype(jnp.int32).T.reshape(-1)
    w_tm = _sc_gather(word_emb, idx, T * B, Dw)

    const = pl.pallas_call(
        _const_body,
        out_shape=jax.ShapeDtypeStruct((B, 4 * Dh), jnp.float32),
    )(
        object_categories.astype(jnp.int32),
        num_objects.astype(jnp.float32).reshape(B, 1),
        category_emb,
        visual_features,
        W_vis,
        b_vis.reshape(1, Dh),
        W_ih[Dw:],
        (b_ih + b_hh).reshape(1, 4 * Dh),
    )

    grid = (NT,)
    out = pl.pallas_call(
        _main_body,
        grid=grid,
        in_specs=[
            pl.BlockSpec((TB * B, Dw), lambda i: (i, 0)),
            pl.BlockSpec((Dw, 4 * Dh), lambda i: (0, 0)),
            pl.BlockSpec((Dh, 4 * Dh), lambda i: (0, 0)),
            pl.BlockSpec((B, 4 * Dh), lambda i: (0, 0)),
            pl.BlockSpec((Dh, V), lambda i: (0, 0)),
            pl.BlockSpec((1, V), lambda i: (0, 0)),
        ],
        out_specs=pl.BlockSpec((B, TB, V), lambda i: (0, i, 0)),
        out_shape=jax.ShapeDtypeStruct((B, T, V), jnp.float32),
        scratch_shapes=[
            pltpu.VMEM((TB * B, 4 * Dh), jnp.float32),
            pltpu.VMEM((TB * B, Dh), jnp.bfloat16),
            pltpu.VMEM((B, Dh), jnp.float32),
            pltpu.VMEM((B, Dh), jnp.float32),
        ],
    )(w_tm, W_ih[:Dw].astype(jnp.bfloat16), W_hh.astype(jnp.bfloat16),
      const, W_out.astype(jnp.bfloat16), b_out.reshape(1, V))
    return out
